# baseline (speedup 1.0000x reference)
import math

import jax
import jax.numpy as jnp
import numpy as np
from jax import lax
from jax.experimental import pallas as pl
from jax.experimental.pallas import tpu as pltpu
from jax.experimental.pallas import tpu_sc as plsc

VOCAB = 1000000
D = 64
B = 4096
L = 200
N = B * L
NW = 32
SPW = B // NW
NCH = 2 * SPW
CA = 104
CB = L - CA
PELINES = L // 2


def _make_pe():
    pe = np.zeros((L, D), dtype=np.float32)
    position = np.arange(0.0, L, dtype=np.float64)[:, None]
    div_term = np.exp(
        np.arange(0.0, D, 2, dtype=np.float64) * -(math.log(10000.0) / D))
    pe[:, 0::2] = np.sin(position * div_term)
    pe[:, 1::2] = np.cos(position * div_term)
    return pe.reshape(PELINES, 2 * D)


_PE_PACKED = _make_pe()

_SCALE = math.sqrt(D)


def _emb_body(x_hbm, pe_hbm, lut_hbm, out_hbm,
              idx_v, pe_v, r0, r1, o0, o1, x0, x1,
              g0, g1, s0, s1):
    rows = [r0, r1]
    outc = [o0, o1]
    gidx = [x0, x1]
    gsem = [g0, g1]
    osem = [s0, s1]

    cid = lax.axis_index("c")
    sid = lax.axis_index("s")
    wid = sid * 2 + cid
    bbase = wid * SPW

    pltpu.sync_copy(x_hbm.at[wid], idx_v)
    pltpu.sync_copy(pe_hbm, pe_v)

    def gen_idx(c, slot, nblk):
        def blk(b16, carry):
            sl = pl.ds(b16 * 16, 16)
            gidx[slot][sl] = lax.shift_right_logical(idx_v[c, sl], 1)
            return carry
        lax.fori_loop(0, nblk, blk, 0)

    def issue_gather(slot, n):
        pltpu.async_copy(
            lut_hbm.at[gidx[slot].at[pl.ds(0, n)]],
            rows[slot].at[pl.ds(0, n)], gsem[slot])

    def wait_gather(slot, n):
        pltpu.make_async_copy(
            lut_hbm.at[gidx[slot].at[pl.ds(0, n)]],
            rows[slot].at[pl.ds(0, n)], gsem[slot]).wait()

    gen_idx(0, 0, CA // 16 + 1)
    issue_gather(0, CA)

    def body16(c, rv, ov, jb, l0, njj):
        off16 = (idx_v[c, pl.ds(jb, 16)] & 1) * D
        for jj in range(njj):
            j = jb + jj
            off = off16[jj]
            prow = (l0 + jb) // 2 + (jj >> 1)
            pc0 = (jj & 1) * D
            for d in range(D // 16):
                ov[0, j, pl.ds(d * 16, 16)] = (
                    rv[j, pl.ds(off + d * 16, 16)] * _SCALE
                    + pe_v[prow, pl.ds(pc0 + d * 16, 16)])

    def compute_a(c, rv, ov):
        def blk(b16, carry):
            body16(c, rv, ov, b16 * 16, 0, 16)
            return carry
        lax.fori_loop(0, CA // 16, blk, 0)
        body16(c, rv, ov, (CA // 16) * 16, 0, 8)

    def compute_b(c, rv, ov):
        def blk(b16, carry):
            body16(c, rv, ov, b16 * 16, CA, 16)
            return carry
        lax.fori_loop(0, CB // 16, blk, 0)

    def chunk(c, carry):
        r = lax.rem(c, 2)
        s = lax.shift_right_logical(c, 1)

        @pl.when(r == 0)
        def _():
            wait_gather(0, CA)
            gen_idx(c + 1, 1, CB // 16)
            issue_gather(1, CB)

            @pl.when(c >= 2)
            def _():
                pltpu.make_async_copy(
                    outc[0], out_hbm.at[pl.ds(bbase, 1), pl.ds(0, CA)],
                    osem[0]).wait()

            compute_a(c, rows[0], outc[0])
            pltpu.async_copy(
                outc[0], out_hbm.at[pl.ds(bbase + s, 1), pl.ds(0, CA)],
                osem[0])

        @pl.when(r == 1)
        def _():
            wait_gather(1, CB)

            @pl.when(c < NCH - 1)
            def _():
                gen_idx(c + 1, 0, CA // 16 + 1)
                issue_gather(0, CA)

            @pl.when(c >= 2)
            def _():
                pltpu.make_async_copy(
                    outc[1], out_hbm.at[pl.ds(bbase, 1), pl.ds(CA, CB)],
                    osem[1]).wait()

            compute_b(c, rows[1], outc[1])
            pltpu.async_copy(
                outc[1], out_hbm.at[pl.ds(bbase + s, 1), pl.ds(CA, CB)],
                osem[1])

        return carry

    lax.fori_loop(0, NCH, chunk, 0)

    pltpu.make_async_copy(
        outc[0], out_hbm.at[pl.ds(bbase, 1), pl.ds(0, CA)], osem[0]).wait()
    pltpu.make_async_copy(
        outc[1], out_hbm.at[pl.ds(bbase, 1), pl.ds(CA, CB)], osem[1]).wait()


_emb_call = pl.kernel(
    _emb_body,
    out_type=jax.ShapeDtypeStruct((B, L, D), jnp.float32),
    mesh=plsc.VectorSubcoreMesh(core_axis_name="c", subcore_axis_name="s"),
    scratch_types=(
        [pltpu.VMEM((NCH, 128), jnp.int32),
         pltpu.VMEM((PELINES, 2 * D), jnp.float32)]
        + [pltpu.VMEM((CA, 2 * D), jnp.float32),
           pltpu.VMEM((CB, 2 * D), jnp.float32)]
        + [pltpu.VMEM((1, CA, D), jnp.float32),
           pltpu.VMEM((1, CB, D), jnp.float32)]
        + [pltpu.VMEM((CA + 8, ), jnp.int32),
           pltpu.VMEM((CB, ), jnp.int32)]
        + [pltpu.SemaphoreType.DMA for _ in range(4)]
    ),
)


def kernel(x, lut):
    x2 = x.reshape(NW, SPW, L).astype(jnp.int32)
    a = jnp.pad(x2[:, :, :CA], ((0, 0), (0, 0), (0, 128 - CA)))
    b = jnp.pad(x2[:, :, CA:], ((0, 0), (0, 0), (0, 128 - CB)))
    xr = jnp.stack([a, b], axis=2).reshape(NW, NCH, 128)
    lut2 = lut.reshape(VOCAB // 2, 2 * D)
    pe = jnp.asarray(_PE_PACKED)
    return _emb_call(xr, pe, lut2)


# speedup vs baseline: 1.0700x; 1.0700x over previous
import math

import jax
import jax.numpy as jnp
import numpy as np
from jax import lax
from jax.experimental import pallas as pl
from jax.experimental.pallas import tpu as pltpu
from jax.experimental.pallas import tpu_sc as plsc

VOCAB = 1000000
D = 64
B = 4096
L = 200
N = B * L
NW = 32
NPW = N // NW
C = 128
NCH = NPW // C
NB16 = C // 16
H = C // 2
PEROWS = L + C
PELINES = (PEROWS // 2 + 7) // 8 * 8


def _make_pe():
    pe = np.zeros((PEROWS, D), dtype=np.float32)
    position = np.arange(0.0, PEROWS, dtype=np.float64)[:, None] % L
    div_term = np.exp(
        np.arange(0.0, D, 2, dtype=np.float64) * -(math.log(10000.0) / D))
    pe[:, 0::2] = np.sin(position * div_term)
    pe[:, 1::2] = np.cos(position * div_term)
    out = np.zeros((PELINES, 2 * D), dtype=np.float32)
    out.reshape(-1)[: PEROWS * D] = pe.reshape(-1)
    return out


_PE_PACKED = _make_pe()

_SCALE = math.sqrt(D)


def _emb_body(x_hbm, pe_hbm, lut_hbm, out_hbm,
              idx_v, pe_v, r0, r1, o0, o1, x0, x1,
              g0, g1, s0, s1):
    rows = [r0, r1]
    outc = [o0, o1]
    gidx = [x0, x1]
    gsem = [g0, g1]
    osem = [s0, s1]

    cid = lax.axis_index("c")
    sid = lax.axis_index("s")
    wid = sid * 2 + cid
    obase = wid * NPW

    pltpu.sync_copy(x_hbm.at[wid], idx_v)
    pltpu.sync_copy(pe_hbm, pe_v)

    def make_gidx(cc, slot):
        def blk(b16, carry):
            sl = pl.ds(b16 * 16, 16)
            gidx[slot][sl] = lax.shift_right_logical(idx_v[cc, sl], 1)
            return carry
        lax.fori_loop(0, NB16, blk, 0)

    def issue_gather(slot):
        pltpu.async_copy(
            lut_hbm.at[gidx[slot].at[pl.ds(0, H)]],
            rows[slot].at[pl.ds(0, H)], gsem[slot])
        pltpu.async_copy(
            lut_hbm.at[gidx[slot].at[pl.ds(H, H)]],
            rows[slot].at[pl.ds(H, H)], gsem[slot])

    def wait_gather(slot):
        pltpu.make_async_copy(
            lut_hbm.at[gidx[slot].at[pl.ds(0, H)]],
            rows[slot].at[pl.ds(0, H)], gsem[slot]).wait()
        pltpu.make_async_copy(
            lut_hbm.at[gidx[slot].at[pl.ds(H, H)]],
            rows[slot].at[pl.ds(H, H)], gsem[slot]).wait()

    make_gidx(0, 0)
    issue_gather(0)

    def compute(c, rv, ov):
        ph = lax.rem(c * C, L)
        phh = lax.shift_right_logical(ph, 1)

        def blk(b16, carry2):
            jb = b16 * 16
            sl = pl.ds(jb, 16)
            off16 = (idx_v[c, sl] & 1) * D
            for jj in range(16):
                j = jb + jj
                off = off16[jj]
                prow = phh + b16 * 8 + (jj >> 1)
                pc0 = (jj & 1) * D
                for d in range(D // 16):
                    ov[j, pl.ds(d * 16, 16)] = (
                        rv[j, pl.ds(off + d * 16, 16)] * _SCALE
                        + pe_v[prow, pl.ds(pc0 + d * 16, 16)])
            return carry2

        lax.fori_loop(0, NB16, blk, 0)

    def chunk(c, carry):
        r = lax.rem(c, 2)

        @pl.when(r == 0)
        def _():
            wait_gather(0)

            @pl.when(c < NCH - 1)
            def _():
                make_gidx(c + 1, 1)
                issue_gather(1)

            @pl.when(c >= 2)
            def _():
                pltpu.make_async_copy(
                    outc[0], out_hbm.at[pl.ds(obase, C)], osem[0]).wait()

            compute(c, rows[0], outc[0])
            pltpu.async_copy(
                outc[0], out_hbm.at[pl.ds(obase + c * C, C)], osem[0])

        @pl.when(r == 1)
        def _():
            wait_gather(1)

            @pl.when(c < NCH - 1)
            def _():
                make_gidx(c + 1, 0)
                issue_gather(0)

            @pl.when(c >= 2)
            def _():
                pltpu.make_async_copy(
                    outc[1], out_hbm.at[pl.ds(obase, C)], osem[1]).wait()

            compute(c, rows[1], outc[1])
            pltpu.async_copy(
                outc[1], out_hbm.at[pl.ds(obase + c * C, C)], osem[1])

        return carry

    lax.fori_loop(0, NCH, chunk, 0)

    for b in range(2):
        pltpu.make_async_copy(
            outc[b], out_hbm.at[pl.ds(obase, C)], osem[b]).wait()


_emb_call = pl.kernel(
    _emb_body,
    out_type=jax.ShapeDtypeStruct((N, D), jnp.float32),
    mesh=plsc.VectorSubcoreMesh(core_axis_name="c", subcore_axis_name="s"),
    scratch_types=(
        [pltpu.VMEM((NCH, C), jnp.int32),
         pltpu.VMEM((PELINES, 2 * D), jnp.float32)]
        + [pltpu.VMEM((C, 2 * D), jnp.float32) for _ in range(2)]
        + [pltpu.VMEM((C, D), jnp.float32) for _ in range(2)]
        + [pltpu.VMEM((C,), jnp.int32) for _ in range(2)]
        + [pltpu.SemaphoreType.DMA for _ in range(4)]
    ),
)


def kernel(x, lut):
    xr = x.reshape(NW, NCH, C).astype(jnp.int32)
    lut2 = lut.reshape(VOCAB // 2, 2 * D)
    pe = jnp.asarray(_PE_PACKED)
    out = _emb_call(xr, pe, lut2)
    return out.reshape(B, L, D)
